# Initial kernel scaffold; baseline (speedup 1.0000x reference)
import jax
import jax.numpy as jnp
from jax.experimental import pallas as pl


def kernel(x, edge_index, pos, cell_type, edge_type, edge_weight, params):
    raise NotImplementedError("write your pallas kernel here")


# speedup vs baseline: 1.0078x; 1.0078x over previous
import functools

import jax
import jax.numpy as jnp
import numpy as np
from jax.experimental import pallas as pl
from jax.experimental.pallas import tpu as pltpu

N = 10000
E = 160000
NG = 128
HID = 256
HEADS = 8
HD = 32
NL = 3
NCT = 10
NET = 2

ROWS = 1000


def _enc_body(x_ref, w1_ref, b1_ref, w2_ref, b2_ref, w3_ref, b3_ref, out_ref):
    h = jnp.maximum(x_ref[...] @ w1_ref[...] + b1_ref[...], 0.0)
    h = jnp.maximum(h @ w2_ref[...] + b2_ref[...], 0.0)
    out_ref[...] = h @ w3_ref[...] + b3_ref[...]


def _encoder(x, p):
    grid = (N // ROWS,)
    return pl.pallas_call(
        _enc_body,
        grid=grid,
        in_specs=[
            pl.BlockSpec((ROWS, NG), lambda i: (i, 0)),
            pl.BlockSpec((NG, 512), lambda i: (0, 0)),
            pl.BlockSpec((512,), lambda i: (0,)),
            pl.BlockSpec((512, 256), lambda i: (0, 0)),
            pl.BlockSpec((256,), lambda i: (0,)),
            pl.BlockSpec((256, HID), lambda i: (0, 0)),
            pl.BlockSpec((HID,), lambda i: (0,)),
        ],
        out_specs=pl.BlockSpec((ROWS, HID), lambda i: (i, 0)),
        out_shape=jax.ShapeDtypeStruct((N, HID), jnp.float32),
    )(x, p['enc_W1'], p['enc_b1'], p['enc_W2'], p['enc_b2'], p['enc_W3'], p['enc_b3'])


def _sinus(pos, dim=64):
    nfreq = dim // (2 * pos.shape[1])
    freqs = jnp.exp(jnp.arange(nfreq, dtype=jnp.float32) * (-np.log(10000.0) / max(nfreq - 1, 1)))
    ang = pos[:, :, None] * freqs[None, None, :]
    enc = jnp.concatenate([jnp.sin(ang), jnp.cos(ang)], axis=-1)
    return enc.reshape(pos.shape[0], -1)


def kernel(x, edge_index, pos, cell_type, edge_type, edge_weight, params):
    p = params
    src, dst = edge_index[0], edge_index[1]
    gz = _encoder(x, p)
    sp = _sinus(pos, 64)
    ct = p['ct_emb'][cell_type]
    z = jnp.concatenate([gz, sp, ct], axis=-1) @ p['fuse_W'] + p['fuse_b']
    layer_outs = []
    for l in range(NL):
        hh = (z @ p['gat_W%d' % l]).reshape(N, HEADS, HD)
        a_s = jnp.sum(hh * p['gat_as%d' % l][None], axis=-1)
        a_d = jnp.sum(hh * p['gat_ad%d' % l][None], axis=-1)
        e = jax.nn.leaky_relu(a_s[src] + a_d[dst] + p['gat_et%d' % l][edge_type], 0.2)
        m = jax.ops.segment_max(e, dst, num_segments=N)
        m = jax.lax.stop_gradient(jnp.where(jnp.isfinite(m), m, 0.0))
        ex = jnp.exp(e - m[dst])
        s = jax.ops.segment_sum(ex, dst, num_segments=N)
        attn = ex / (s[dst] + 1e-16)
        attn = attn * edge_weight[:, None]
        msg = attn[:, :, None] * hh[src]
        agg = jax.ops.segment_sum(msg, dst, num_segments=N).reshape(N, HID) + p['gat_b%d' % l]
        z = jax.nn.elu(agg)
        layer_outs.append(z)
    zj = jnp.concatenate(layer_outs, axis=-1)
    z_gat = zj @ p['jk_W'] + p['jk_b']
    dist = jnp.sqrt(jnp.sum((pos[src] - pos[dst]) ** 2, axis=-1) + 1e-12)
    ef = jnp.concatenate([z_gat[src], z_gat[dst], dist[:, None]], axis=-1)
    eh = jax.nn.relu(ef @ p['lr_W1'] + p['lr_b1'])
    lr_scores = (eh @ p['lr_W2'] + p['lr_b2'])[:, 0]
    d1 = jax.nn.relu(z_gat @ p['dec_W1'] + p['dec_b1'])
    recon = d1 @ p['dec_W2'] + p['dec_b2'] + z_gat @ p['dec_skip']
    return lr_scores, recon, z_gat


# speedup vs baseline: 20.8647x; 20.7033x over previous
import functools

import jax
import jax.numpy as jnp
import numpy as np
from jax import lax
from jax.experimental import pallas as pl
from jax.experimental.pallas import tpu as pltpu
from jax.experimental.pallas import tpu_sc as plsc

N = 10000
E = 160000
NG = 128
HID = 256
HEADS = 8
HD = 32
NL = 3
NCT = 10
NET = 2

ROWS = 1000


def _enc_body(x_ref, w1_ref, b1_ref, w2_ref, b2_ref, w3_ref, b3_ref, out_ref):
    h = jnp.maximum(x_ref[...] @ w1_ref[...] + b1_ref[...], 0.0)
    h = jnp.maximum(h @ w2_ref[...] + b2_ref[...], 0.0)
    out_ref[...] = h @ w3_ref[...] + b3_ref[...]


def _encoder(x, p):
    grid = (N // ROWS,)
    return pl.pallas_call(
        _enc_body,
        grid=grid,
        in_specs=[
            pl.BlockSpec((ROWS, NG), lambda i: (i, 0)),
            pl.BlockSpec((NG, 512), lambda i: (0, 0)),
            pl.BlockSpec((512,), lambda i: (0,)),
            pl.BlockSpec((512, 256), lambda i: (0, 0)),
            pl.BlockSpec((256,), lambda i: (0,)),
            pl.BlockSpec((256, HID), lambda i: (0, 0)),
            pl.BlockSpec((HID,), lambda i: (0,)),
        ],
        out_specs=pl.BlockSpec((ROWS, HID), lambda i: (i, 0)),
        out_shape=jax.ShapeDtypeStruct((N, HID), jnp.float32),
    )(x, p['enc_W1'], p['enc_b1'], p['enc_W2'], p['enc_b2'], p['enc_W3'], p['enc_b3'])


C = 80
EPT = E // 32
NCH = EPT // C
NP = 10240
STRIPE = NP // 16


def _make_sc_gat_body(ntec):
    return functools.partial(_sc_gat_body_impl, ntec)


def _sc_gat_body_impl(ntec, hh_ref, as_ref, ad_ref, src_ref, dst_ref, et_ref, ew_ref,
                 e00_ref, e01_ref, e10_ref, e11_ref, z128_ref, z16_ref,
                 agg_out, s_out,
                 agg_sh, s_sh, src_b, dst_b, et_b, ew_b, rows, asg, adg, sxb, wxb,
                 b00, b01, b10, b11, sem0, sem1, sem2):
    c = lax.axis_index("c")
    t = lax.axis_index("s")
    t0 = t * STRIPE
    pltpu.sync_copy(z128_ref.at[pl.ds(t0, STRIPE)], agg_sh.at[pl.ds(t0, STRIPE)])
    pltpu.sync_copy(z16_ref.at[pl.ds(t0, STRIPE)], s_sh.at[pl.ds(t0, STRIPE)])
    pltpu.sync_copy(e00_ref, b00)
    pltpu.sync_copy(e01_ref, b01)
    pltpu.sync_copy(e10_ref, b10)
    pltpu.sync_copy(e11_ref, b11)
    plsc.subcore_barrier()
    cf = jnp.full((16,), c, jnp.float32)
    b0v = b00[...] + cf * (b10[...] - b00[...])
    b1v = b01[...] + cf * (b11[...] - b01[...])
    bdv = b1v - b0v

    ept = E // ntec

    def chunk(i, _):
        base = t * ept + i * C
        pltpu.sync_copy(src_ref.at[pl.ds(base, C)], src_b)
        pltpu.sync_copy(dst_ref.at[pl.ds(base, C)], dst_b)
        pltpu.sync_copy(et_ref.at[pl.ds(base, C)], et_b)
        pltpu.sync_copy(ew_ref.at[pl.ds(base, C)], ew_b)
        cp0 = pltpu.async_copy(hh_ref.at[c].at[src_b], rows, sem0)
        cp1 = pltpu.async_copy(as_ref.at[c].at[src_b], asg, sem1)
        cp2 = pltpu.async_copy(ad_ref.at[c].at[dst_b], adg, sem2)
        cp1.wait()
        cp2.wait()

        def p1(g, _):
            et16 = et_b[pl.ds(g * 16, 16)]
            ew16 = ew_b[pl.ds(g * 16, 16)]
            etf = (et16 * 1).astype(jnp.float32)
            for j2 in range(16):
                k = g * 16 + j2
                asv = asg[k, :]
                adv = adg[k, :]
                bsel = b0v + jnp.full((16,), etf[j2], jnp.float32) * bdv
                e = asv + adv + bsel
                e = jnp.maximum(e, 0.2 * e)
                ex = jnp.exp(e)
                sxb[k, :] = ex
                wxb[k, :] = ex * jnp.full((16,), ew16[j2], jnp.float32)
            return 0

        lax.fori_loop(0, C // 16, p1, 0)
        cp0.wait()

        def p2(k, _):
            wrow = wxb[k, :]
            for h in range(4):
                wh = jnp.full((16,), wrow[h], jnp.float32)
                for v in range(2):
                    sl = pl.ds(32 * h + 16 * v, 16)
                    rows[k, sl] = rows[k, sl] * wh
            return 0

        lax.fori_loop(0, C, p2, 0)
        pltpu.sync_copy(rows, agg_sh.at[dst_b], add=True)
        pltpu.sync_copy(sxb, s_sh.at[dst_b], add=True)
        return 0

    @pl.when(t < ntec)
    def _():
        lax.fori_loop(0, ept // C, chunk, 0)
    plsc.subcore_barrier()
    pltpu.sync_copy(agg_sh.at[pl.ds(t0, STRIPE)], agg_out.at[c].at[pl.ds(t0, STRIPE)])
    pltpu.sync_copy(s_sh.at[pl.ds(t0, STRIPE)], s_out.at[c].at[pl.ds(t0, STRIPE)])


_sc_gat_out_type = (jax.ShapeDtypeStruct((2, NP, 128), jnp.float32),
                    jax.ShapeDtypeStruct((2, NP, 16), jnp.float32))
_sc_gat_mesh = plsc.VectorSubcoreMesh(core_axis_name="c", subcore_axis_name="s")
_sc_gat_scratch = [
    pltpu.VMEM_SHARED((NP, 128), jnp.float32),
    pltpu.VMEM_SHARED((NP, 16), jnp.float32),
    pltpu.VMEM((C,), jnp.int32),
    pltpu.VMEM((C,), jnp.int32),
    pltpu.VMEM((C,), jnp.int32),
    pltpu.VMEM((C,), jnp.float32),
    pltpu.VMEM((C, 128), jnp.float32),
    pltpu.VMEM((C, 16), jnp.float32),
    pltpu.VMEM((C, 16), jnp.float32),
    pltpu.VMEM((C, 16), jnp.float32),
    pltpu.VMEM((C, 16), jnp.float32),
    pltpu.VMEM((16,), jnp.float32),
    pltpu.VMEM((16,), jnp.float32),
    pltpu.VMEM((16,), jnp.float32),
    pltpu.VMEM((16,), jnp.float32),
    pltpu.SemaphoreType.DMA,
    pltpu.SemaphoreType.DMA,
    pltpu.SemaphoreType.DMA,
]
_sc_gat = pl.kernel(
    _make_sc_gat_body(16),
    out_type=_sc_gat_out_type,
    mesh=_sc_gat_mesh,
    compiler_params=pltpu.CompilerParams(use_tc_tiling_on_sc=False),
    scratch_types=_sc_gat_scratch,
)


def _sinus(pos, dim=64):
    nfreq = dim // (2 * pos.shape[1])
    freqs = jnp.exp(jnp.arange(nfreq, dtype=jnp.float32) * (-np.log(10000.0) / max(nfreq - 1, 1)))
    ang = pos[:, :, None] * freqs[None, None, :]
    enc = jnp.concatenate([jnp.sin(ang), jnp.cos(ang)], axis=-1)
    return enc.reshape(pos.shape[0], -1)


def kernel(x, edge_index, pos, cell_type, edge_type, edge_weight, params):
    p = params
    src, dst = edge_index[0], edge_index[1]
    gz = _encoder(x, p)
    sp = _sinus(pos, 64)
    ct = p['ct_emb'][cell_type]
    z = jnp.concatenate([gz, sp, ct], axis=-1) @ p['fuse_W'] + p['fuse_b']
    z128 = jnp.zeros((NP, 128), jnp.float32)
    z16 = jnp.zeros((NP, 16), jnp.float32)
    layer_outs = []
    for l in range(NL):
        W = p['gat_W%d' % l]
        Ws = jnp.einsum('khd,hd->kh', W.reshape(HID, HEADS, HD), p['gat_as%d' % l])
        Wd = jnp.einsum('khd,hd->kh', W.reshape(HID, HEADS, HD), p['gat_ad%d' % l])
        hh = z @ W
        a_s = z @ Ws
        a_d = z @ Wd
        hh2 = hh.reshape(N, 2, 128).transpose(1, 0, 2)
        et = p['gat_et%d' % l]
        lk = jnp.arange(16) & 3
        e00, e01 = et[0, lk], et[1, lk]
        e10, e11 = et[0, 4 + lk], et[1, 4 + lk]
        as16 = jnp.pad(a_s.reshape(N, 2, 4).transpose(1, 0, 2), ((0, 0), (0, 0), (0, 12)))
        ad16 = jnp.pad(a_d.reshape(N, 2, 4).transpose(1, 0, 2), ((0, 0), (0, 0), (0, 12)))
        aggU, sseg = _sc_gat(hh2, as16, ad16, src, dst, edge_type, edge_weight,
                             e00, e01, e10, e11, z128, z16)
        agg = aggU[:, :N].transpose(1, 0, 2).reshape(N, HID)
        srep = jnp.repeat(sseg[:, :N, :4].transpose(1, 0, 2).reshape(N, HEADS), HD, axis=1)
        z = jax.nn.elu(agg / (srep + 1e-16) + p['gat_b%d' % l])
        layer_outs.append(z)
    zj = jnp.concatenate(layer_outs, axis=-1)
    z_gat = zj @ p['jk_W'] + p['jk_b']
    dist = jnp.sqrt(jnp.sum((pos[src] - pos[dst]) ** 2, axis=-1) + 1e-12)
    ef = jnp.concatenate([z_gat[src], z_gat[dst], dist[:, None]], axis=-1)
    eh = jax.nn.relu(ef @ p['lr_W1'] + p['lr_b1'])
    lr_scores = (eh @ p['lr_W2'] + p['lr_b2'])[:, 0]
    d1 = jax.nn.relu(z_gat @ p['dec_W1'] + p['dec_b1'])
    recon = d1 @ p['dec_W2'] + p['dec_b2'] + z_gat @ p['dec_skip']
    return lr_scores, recon, z_gat
